# baseline (speedup 1.0000x reference)
import functools

import jax
import jax.numpy as jnp
from jax import lax
from jax.experimental import pallas as pl
from jax.experimental.pallas import tpu as pltpu
from jax.experimental.pallas import tpu_sc as plsc

N_NODES = 10000
N_EDGES = 160000
FEAT = 256
FPAD = 272
NSC = 2
NSUB = 16
NHALF = N_NODES // NSC
EPT = 10240
EPAD = NSUB * EPT
CHUNK = 16
NBUF = 5
IDXB = EPT + NBUF * CHUNK + 16
ZROWS = 320
AGG_ROWS = NSUB * ZROWS
DUMMY_SRC = N_NODES
DUMMY_DST = NHALF


def _sc_aggregate(xfull, src_p, dst_p, zrows):
  mesh = plsc.VectorSubcoreMesh(
      core_axis_name="c", subcore_axis_name="s",
      num_cores=NSC, num_subcores=NSUB)

  @functools.partial(
      pl.kernel,
      out_type=[jax.ShapeDtypeStruct((NSC, AGG_ROWS, 128), jnp.float32)] * 3,
      mesh=mesh,
      compiler_params=pltpu.CompilerParams(
          use_tc_tiling_on_sc=False, needs_layout_passes=False),
      scratch_types=[
          pltpu.VMEM_SHARED((AGG_ROWS, FPAD), jnp.float32),
          pltpu.VMEM((IDXB,), jnp.int32),
          pltpu.VMEM((IDXB,), jnp.int32),
          [pltpu.VMEM((CHUNK, FPAD), jnp.float32)] * NBUF,
          [pltpu.SemaphoreType.DMA] * NBUF,
      ],
  )
  def agg_kernel(xfull_hbm, src_hbm, dst_hbm, z_hbm, out0_hbm, outm_hbm,
                 out1_hbm, acc, srcb, dstb, rows, sems):
    c = lax.axis_index("c")
    s = lax.axis_index("s")
    lo = c * NHALF

    pltpu.sync_copy(z_hbm, acc.at[pl.ds(s * ZROWS, ZROWS)])

    pltpu.sync_copy(src_hbm.at[s], srcb.at[pl.ds(0, EPT)])
    pltpu.sync_copy(dst_hbm.at[s], dstb.at[pl.ds(0, EPT)])

    @pl.loop(0, EPT // 16, init_carry=jnp.int32(0))
    def _filter(i, off):
      sl = pl.ds(i * 16, 16)
      d = dstb[sl]
      sv = srcb[sl]
      msk = (d >= lo) & (d < lo + NHALF)
      plsc.store_compressed(dstb.at[pl.ds(off, 16)], d - lo, mask=msk)
      plsc.store_compressed(srcb.at[pl.ds(off, 16)], sv, mask=msk)
      cnt = plsc.all_reduce_population_count(msk)
      return off + cnt[0]

    m = _filter
    base = pl.multiple_of((m // 16) * 16, 16)
    lanes = lax.iota(jnp.int32, 16)
    keep = lanes < (m - base)
    bsl = pl.ds(base, 16)
    srcb[bsl] = jnp.where(keep, srcb[bsl], DUMMY_SRC)
    dstb[bsl] = jnp.where(keep, dstb[bsl], DUMMY_DST)
    for k in range(1, NBUF * CHUNK // 16 + 1):
      sl = pl.ds(base + k * 16, 16)
      srcb[sl] = jnp.full((16,), DUMMY_SRC, jnp.int32)
      dstb[sl] = jnp.full((16,), DUMMY_DST, jnp.int32)
    grp = NBUF * CHUNK
    nch = jnp.maximum(NBUF * ((m + grp - 1) // grp), NBUF)

    plsc.subcore_barrier()

    def _gather(t, k):
      sl = srcb.at[pl.ds(t * CHUNK, CHUNK)]
      pltpu.async_copy(xfull_hbm.at[sl], rows[k], sems[k])

    for k in range(NBUF - 1):
      _gather(k, k)

    @pl.loop(0, nch, step=NBUF)
    def _(j):
      for k in range(NBUF):
        t = j + k
        sl = srcb.at[pl.ds(t * CHUNK, CHUNK)]
        pltpu.make_async_copy(xfull_hbm.at[sl], rows[k], sems[k]).wait()

        @pl.when(t + NBUF - 1 < nch)
        def _():
          _gather(t + NBUF - 1, (k + NBUF - 1) % NBUF)

        dv = dstb[pl.ds(t * CHUNK, 16)]
        pltpu.sync_copy(rows[k], acc.at[dv], add=True)

    plsc.subcore_barrier()
    rsl = pl.ds(s * ZROWS, ZROWS)
    pltpu.sync_copy(acc.at[rsl, pl.ds(0, 128)], out0_hbm.at[c, rsl])
    pltpu.sync_copy(acc.at[rsl, pl.ds(128, 128)], outm_hbm.at[c, rsl])
    pltpu.sync_copy(acc.at[rsl, pl.ds(136, 128)], out1_hbm.at[c, rsl])

  return agg_kernel(xfull, src_p, dst_p, zrows)


def _tc_body(x_ref, a0_ref, am_ref, a1_ref, w_ref, lw_ref, b_ref, out_ref):
  deg = jnp.maximum(am_ref[0][:, 0:1], 1.0)
  inv = 1.0 / deg
  acc = jnp.dot(a0_ref[0] * inv, w_ref[:128, :],
                preferred_element_type=jnp.float32)
  acc = acc + jnp.dot(a1_ref[0] * inv, w_ref[128:, :],
                      preferred_element_type=jnp.float32)
  acc = acc + jnp.dot(x_ref[...], lw_ref[...], preferred_element_type=jnp.float32)
  out_ref[...] = acc + b_ref[...]


def _tc_combine(x, a0, am, a1, w, lw, b2):
  nblk = 10
  blk = N_NODES // nblk
  bph = NHALF // blk
  strip = pl.BlockSpec((1, blk, 128), lambda i: (i // bph, i % bph, 0))
  return pl.pallas_call(
      _tc_body,
      grid=(nblk,),
      in_specs=[
          pl.BlockSpec((blk, FEAT), lambda i: (i, 0)),
          strip, strip, strip,
          pl.BlockSpec((FEAT, FEAT), lambda i: (0, 0)),
          pl.BlockSpec((FEAT, FEAT), lambda i: (0, 0)),
          pl.BlockSpec((1, FEAT), lambda i: (0, 0)),
      ],
      out_specs=pl.BlockSpec((blk, FEAT), lambda i: (i, 0)),
      out_shape=jax.ShapeDtypeStruct((N_NODES, FEAT), jnp.float32),
  )(x, a0, am, a1, w, lw, b2)


def kernel(x, edge_index, W, b, loop_weight):
  ei = edge_index.astype(jnp.int32)
  src = ei[0]
  dst = ei[1]
  pad = EPAD - N_EDGES
  src_p = jnp.concatenate([src, jnp.full((pad,), DUMMY_SRC, jnp.int32)])
  src_p = src_p.reshape(NSUB, EPT)
  dst_p = jnp.concatenate([dst, jnp.full((pad,), N_NODES, jnp.int32)])
  dst_p = dst_p.reshape(NSUB, EPT)

  ones = jnp.ones((N_NODES, 1), jnp.float32)
  z7 = jnp.zeros((N_NODES, 7), jnp.float32)
  z8 = jnp.zeros((N_NODES, 8), jnp.float32)
  xfull = jnp.concatenate([x[:, :128], ones, z7, x[:, 128:], z8], axis=1)
  xfull = jnp.concatenate([xfull, jnp.zeros((8, FPAD), jnp.float32)], axis=0)
  zrows = jnp.zeros((ZROWS, FPAD), jnp.float32)

  a0, am, a1 = _sc_aggregate(xfull, src_p, dst_p, zrows)

  return _tc_combine(x, a0, am, a1, W, loop_weight, b.reshape(1, FEAT))


# speedup vs baseline: 1.1172x; 1.1172x over previous
import functools

import jax
import jax.numpy as jnp
from jax import lax
from jax.experimental import pallas as pl
from jax.experimental.pallas import tpu as pltpu
from jax.experimental.pallas import tpu_sc as plsc

N_NODES = 10000
N_EDGES = 160000
FEAT = 256
FPAD = 272
NSC = 2
NSUB = 16
NHALF = N_NODES // NSC
EPT = 10240
EPAD = NSUB * EPT
CHUNK = 16
NBUF = 5
IDXB = EPT + NBUF * CHUNK + 16
ZROWS = 316
AGG_ROWS = NSUB * ZROWS
DUMMY_SRC = N_NODES
DUMMY_DST = NHALF


def _sc_aggregate(xfull, e_p, zrows):
  mesh = plsc.VectorSubcoreMesh(
      core_axis_name="c", subcore_axis_name="s",
      num_cores=NSC, num_subcores=NSUB)

  @functools.partial(
      pl.kernel,
      out_type=jax.ShapeDtypeStruct((NSC, AGG_ROWS, FPAD), jnp.float32),
      mesh=mesh,
      compiler_params=pltpu.CompilerParams(
          use_tc_tiling_on_sc=False, needs_layout_passes=False),
      scratch_types=[
          pltpu.VMEM_SHARED((AGG_ROWS, FPAD), jnp.float32),
          pltpu.VMEM((IDXB,), jnp.int32),
          pltpu.VMEM((IDXB,), jnp.int32),
          [pltpu.VMEM((CHUNK, FPAD), jnp.float32)] * NBUF,
          [pltpu.SemaphoreType.DMA] * NBUF,
      ],
  )
  def agg_kernel(xfull_hbm, e_hbm, z_hbm, out_hbm,
                 acc, srcb, dstb, rows, sems):
    c = lax.axis_index("c")
    s = lax.axis_index("s")
    lo = c * NHALF

    pltpu.sync_copy(z_hbm, acc.at[pl.ds(s * ZROWS, ZROWS)])

    pltpu.sync_copy(e_hbm.at[0, s], srcb.at[pl.ds(0, EPT)])
    pltpu.sync_copy(e_hbm.at[1, s], dstb.at[pl.ds(0, EPT)])

    @pl.loop(0, EPT // 16, init_carry=jnp.int32(0))
    def _filter(i, off):
      sl = pl.ds(i * 16, 16)
      d = dstb[sl]
      sv = srcb[sl]
      msk = (d >= lo) & (d < lo + NHALF)
      plsc.store_compressed(dstb.at[pl.ds(off, 16)], d - lo, mask=msk)
      plsc.store_compressed(srcb.at[pl.ds(off, 16)], sv, mask=msk)
      cnt = plsc.all_reduce_population_count(msk)
      return off + cnt[0]

    m = _filter
    base = pl.multiple_of((m // 16) * 16, 16)
    lanes = lax.iota(jnp.int32, 16)
    keep = lanes < (m - base)
    bsl = pl.ds(base, 16)
    srcb[bsl] = jnp.where(keep, srcb[bsl], DUMMY_SRC)
    dstb[bsl] = jnp.where(keep, dstb[bsl], DUMMY_DST)
    for k in range(1, NBUF * CHUNK // 16 + 1):
      sl = pl.ds(base + k * 16, 16)
      srcb[sl] = jnp.full((16,), DUMMY_SRC, jnp.int32)
      dstb[sl] = jnp.full((16,), DUMMY_DST, jnp.int32)
    grp = NBUF * CHUNK
    nch = jnp.maximum(NBUF * ((m + grp - 1) // grp), NBUF)

    plsc.subcore_barrier()

    def _gather(t, k):
      sl = srcb.at[pl.ds(t * CHUNK, CHUNK)]
      pltpu.async_copy(xfull_hbm.at[sl], rows[k], sems[k])

    for k in range(NBUF - 1):
      _gather(k, k)

    @pl.loop(0, nch, step=NBUF)
    def _(j):
      for k in range(NBUF):
        t = j + k
        sl = srcb.at[pl.ds(t * CHUNK, CHUNK)]
        pltpu.make_async_copy(xfull_hbm.at[sl], rows[k], sems[k]).wait()

        @pl.when(t + NBUF - 1 < nch)
        def _():
          _gather(t + NBUF - 1, (k + NBUF - 1) % NBUF)

        dv = dstb[pl.ds(t * CHUNK, 16)]
        pltpu.sync_copy(rows[k], acc.at[dv], add=True)

    plsc.subcore_barrier()
    pltpu.sync_copy(acc.at[pl.ds(s * ZROWS, ZROWS)],
                    out_hbm.at[c, pl.ds(s * ZROWS, ZROWS)])

  return agg_kernel(xfull, e_p, zrows)


def _tc_body(x_ref, agg_ref, w_ref, lw_ref, b_ref, out_ref):
  a = agg_ref[0]
  deg = jnp.maximum(a[:, FEAT:FEAT + 1], 1.0)
  inv = 1.0 / deg
  acc = jnp.dot(a[:, :FEAT] * inv, w_ref[...],
                preferred_element_type=jnp.float32)
  acc = acc + jnp.dot(x_ref[...], lw_ref[...], preferred_element_type=jnp.float32)
  out_ref[...] = acc + b_ref[...]


def _tc_combine(x, agg, w2p, lw, b2):
  nblk = 10
  blk = N_NODES // nblk
  bph = NHALF // blk
  return pl.pallas_call(
      _tc_body,
      grid=(nblk,),
      in_specs=[
          pl.BlockSpec((blk, FEAT), lambda i: (i, 0)),
          pl.BlockSpec((1, blk, FPAD), lambda i: (i // bph, i % bph, 0)),
          pl.BlockSpec((FEAT, FEAT), lambda i: (0, 0)),
          pl.BlockSpec((FEAT, FEAT), lambda i: (0, 0)),
          pl.BlockSpec((1, FEAT), lambda i: (0, 0)),
      ],
      out_specs=pl.BlockSpec((blk, FEAT), lambda i: (i, 0)),
      out_shape=jax.ShapeDtypeStruct((N_NODES, FEAT), jnp.float32),
  )(x, agg, w2p, lw, b2)


def kernel(x, edge_index, W, b, loop_weight):
  ei = edge_index.astype(jnp.int32)
  pad = EPAD - N_EDGES
  e_p = jnp.concatenate(
      [ei, jnp.full((2, pad), N_NODES, jnp.int32)], axis=1)
  e_p = e_p.reshape(2, NSUB, EPT)

  ones = jnp.ones((N_NODES, 1), jnp.float32)
  zcols = jnp.zeros((N_NODES, FPAD - FEAT - 1), jnp.float32)
  xfull = jnp.concatenate([x, ones, zcols], axis=1)
  xfull = jnp.concatenate([xfull, jnp.zeros((8, FPAD), jnp.float32)], axis=0)
  zrows = jnp.zeros((ZROWS, FPAD), jnp.float32)

  agg = _sc_aggregate(xfull, e_p, zrows)

  return _tc_combine(x, agg, W, loop_weight, b.reshape(1, FEAT))


# speedup vs baseline: 1.1264x; 1.0083x over previous
import functools

import jax
import jax.numpy as jnp
from jax import lax
from jax.experimental import pallas as pl
from jax.experimental.pallas import tpu as pltpu
from jax.experimental.pallas import tpu_sc as plsc

N_NODES = 10000
N_EDGES = 160000
FEAT = 256
FPAD = 272
NSC = 2
NSUB = 16
NHALF = N_NODES // NSC
EPT = 10240
EPAD = NSUB * EPT
CHUNK = 16
NBUF = 5
IDXB = EPT + NBUF * CHUNK + 16
ZROWS = 316
AGG_ROWS = NSUB * ZROWS
DUMMY_SRC = N_NODES
DUMMY_DST = NHALF


def _sc_aggregate(xfull, e_p, zrows):
  mesh = plsc.VectorSubcoreMesh(
      core_axis_name="c", subcore_axis_name="s",
      num_cores=NSC, num_subcores=NSUB)

  @functools.partial(
      pl.kernel,
      out_type=jax.ShapeDtypeStruct((NSC, AGG_ROWS, FPAD), jnp.float32),
      mesh=mesh,
      compiler_params=pltpu.CompilerParams(
          use_tc_tiling_on_sc=False, needs_layout_passes=False),
      scratch_types=[
          pltpu.VMEM_SHARED((AGG_ROWS, FPAD), jnp.float32),
          pltpu.VMEM((IDXB,), jnp.int32),
          pltpu.VMEM((IDXB,), jnp.int32),
          [pltpu.VMEM((CHUNK, FPAD), jnp.float32)] * NBUF,
          [pltpu.SemaphoreType.DMA] * NBUF,
      ],
  )
  def agg_kernel(xfull_hbm, e_hbm, z_hbm, out_hbm,
                 acc, srcb, dstb, rows, sems):
    c = lax.axis_index("c")
    s = lax.axis_index("s")
    lo = c * NHALF

    pltpu.sync_copy(z_hbm, acc.at[pl.ds(s * ZROWS, ZROWS)])

    pltpu.sync_copy(e_hbm.at[0, s], srcb.at[pl.ds(0, EPT)])
    pltpu.sync_copy(e_hbm.at[1, s], dstb.at[pl.ds(0, EPT)])

    @pl.loop(0, EPT // 16, init_carry=jnp.int32(0))
    def _filter(i, off):
      sl = pl.ds(i * 16, 16)
      d = dstb[sl]
      sv = srcb[sl]
      msk = (d >= lo) & (d < lo + NHALF)
      plsc.store_compressed(dstb.at[pl.ds(off, 16)], d - lo, mask=msk)
      plsc.store_compressed(srcb.at[pl.ds(off, 16)], sv, mask=msk)
      cnt = plsc.all_reduce_population_count(msk)
      return off + cnt[0]

    m = _filter
    base = pl.multiple_of((m // 16) * 16, 16)
    lanes = lax.iota(jnp.int32, 16)
    keep = lanes < (m - base)
    bsl = pl.ds(base, 16)
    srcb[bsl] = jnp.where(keep, srcb[bsl], DUMMY_SRC)
    dstb[bsl] = jnp.where(keep, dstb[bsl], DUMMY_DST)
    for k in range(1, NBUF * CHUNK // 16 + 1):
      sl = pl.ds(base + k * 16, 16)
      srcb[sl] = jnp.full((16,), DUMMY_SRC, jnp.int32)
      dstb[sl] = jnp.full((16,), DUMMY_DST, jnp.int32)
    grp = NBUF * CHUNK
    nch = jnp.maximum(NBUF * ((m + grp - 1) // grp), NBUF)

    plsc.subcore_barrier()

    def _gather(t, k):
      sl = srcb.at[pl.ds(t * CHUNK, CHUNK)]
      pltpu.async_copy(xfull_hbm.at[sl], rows[k], sems[k])

    for k in range(NBUF - 1):
      _gather(k, k)

    @pl.loop(0, nch, step=NBUF)
    def _(j):
      for k in range(NBUF):
        t = j + k
        sl = srcb.at[pl.ds(t * CHUNK, CHUNK)]
        pltpu.make_async_copy(xfull_hbm.at[sl], rows[k], sems[k]).wait()

        @pl.when(t + NBUF - 1 < nch)
        def _():
          _gather(t + NBUF - 1, (k + NBUF - 1) % NBUF)

        dv = dstb[pl.ds(t * CHUNK, 16)]
        pltpu.sync_copy(rows[k], acc.at[dv], add=True)

    plsc.subcore_barrier()
    pltpu.sync_copy(acc.at[pl.ds(s * ZROWS, ZROWS)],
                    out_hbm.at[c, pl.ds(s * ZROWS, ZROWS)])

  return agg_kernel(xfull, e_p, zrows)


def _tc_body(x_ref, agg_ref, w_ref, lw_ref, b_ref, out_ref):
  a = agg_ref[0]
  deg = jnp.maximum(a[:, FEAT:FEAT + 1], 1.0)
  inv = 1.0 / deg
  acc = jnp.dot(a[:, :FEAT] * inv, w_ref[...],
                preferred_element_type=jnp.float32)
  acc = acc + jnp.dot(x_ref[...], lw_ref[...], preferred_element_type=jnp.float32)
  out_ref[...] = acc + b_ref[...]


def _tc_combine(x, agg, w2p, lw, b2):
  nblk = 5
  blk = N_NODES // nblk
  bph = NHALF // blk
  return pl.pallas_call(
      _tc_body,
      grid=(nblk,),
      in_specs=[
          pl.BlockSpec((blk, FEAT), lambda i: (i, 0)),
          pl.BlockSpec((1, blk, FPAD), lambda i: (i // bph, i % bph, 0)),
          pl.BlockSpec((FEAT, FEAT), lambda i: (0, 0)),
          pl.BlockSpec((FEAT, FEAT), lambda i: (0, 0)),
          pl.BlockSpec((1, FEAT), lambda i: (0, 0)),
      ],
      out_specs=pl.BlockSpec((blk, FEAT), lambda i: (i, 0)),
      out_shape=jax.ShapeDtypeStruct((N_NODES, FEAT), jnp.float32),
  )(x, agg, w2p, lw, b2)


def kernel(x, edge_index, W, b, loop_weight):
  ei = edge_index.astype(jnp.int32)
  pad = EPAD - N_EDGES
  e_p = jnp.concatenate(
      [ei, jnp.full((2, pad), N_NODES, jnp.int32)], axis=1)
  e_p = e_p.reshape(2, NSUB, EPT)

  ones = jnp.ones((N_NODES, 1), jnp.float32)
  zcols = jnp.zeros((N_NODES, FPAD - FEAT - 1), jnp.float32)
  xfull = jnp.concatenate([x, ones, zcols], axis=1)
  xfull = jnp.concatenate([xfull, jnp.zeros((8, FPAD), jnp.float32)], axis=0)
  zrows = jnp.zeros((ZROWS, FPAD), jnp.float32)

  agg = _sc_aggregate(xfull, e_p, zrows)

  return _tc_combine(x, agg, W, loop_weight, b.reshape(1, FEAT))
